# Initial kernel scaffold; baseline (speedup 1.0000x reference)
import jax
import jax.numpy as jnp
from jax.experimental import pallas as pl


def kernel(x, score):
    raise NotImplementedError("write your pallas kernel here")


# speedup vs baseline: 85.9523x; 85.9523x over previous
import functools

import jax
import jax.numpy as jnp
from jax import lax
from jax.experimental import pallas as pl
from jax.experimental.pallas import tpu as pltpu
from jax.experimental.pallas import tpu_sc as plsc

ROWS, COLS = 8192, 4096
TOTAL = ROWS * COLS
NC, NS = 2, 16
NW = NC * NS
PER_W = TOTAL // NW
CHUNK = 16384
NCHUNK = PER_W // CHUNK
GROUPS = CHUNK // 128

_MESH = plsc.VectorSubcoreMesh(core_axis_name="c", subcore_axis_name="s")


def _body(x_hbm, s_hbm, o_hbm, xb, sb, ob):
    wid = lax.axis_index("s") * NC + lax.axis_index("c")
    base_w = wid * PER_W
    vec8 = lax.iota(jnp.int32, 16) * 8

    @pl.loop(0, NCHUNK)
    def _chunk(ci):
        base = base_w + ci * CHUNK
        pltpu.sync_copy(s_hbm.at[pl.ds(base, CHUNK)], sb)
        pltpu.sync_copy(x_hbm.at[pl.ds(base, CHUNK)], xb)

        @pl.loop(0, GROUPS)
        def _group(gi):
            g0 = gi * 128
            idx = [vec8 + (g0 + p) for p in range(8)]
            s = [plsc.load_gather(sb, [idx[p]]) for p in range(8)]
            for p in range(8):
                cnt = jnp.zeros((16,), jnp.int32)
                for q in range(8):
                    if q == p:
                        continue
                    below = (s[q] <= s[p]) if q < p else (s[q] < s[p])
                    cnt = cnt + jnp.where(below, 1, 0)
                keep = cnt >= 4
                xv = plsc.load_gather(xb, [idx[p]])
                plsc.store_scatter(ob, [idx[p]], jnp.where(keep, xv, 0.0))

        pltpu.sync_copy(ob, o_hbm.at[pl.ds(base, CHUNK)])


@jax.jit
def _run(xf, sf):
    return pl.kernel(
        _body,
        out_type=jax.ShapeDtypeStruct((TOTAL,), jnp.float32),
        mesh=_MESH,
        scratch_types=[
            pltpu.VMEM((CHUNK,), jnp.float32),
            pltpu.VMEM((CHUNK,), jnp.float32),
            pltpu.VMEM((CHUNK,), jnp.float32),
        ],
        compiler_params=pltpu.CompilerParams(needs_layout_passes=False),
    )(xf, sf)


def kernel(x, score):
    out = _run(x.reshape(TOTAL), score.reshape(TOTAL))
    return out.reshape(ROWS, COLS)


# speedup vs baseline: 107.3469x; 1.2489x over previous
import jax
import jax.numpy as jnp
from jax import lax
from jax.experimental import pallas as pl
from jax.experimental.pallas import tpu as pltpu
from jax.experimental.pallas import tpu_sc as plsc

ROWS, COLS = 8192, 4096
TOTAL = ROWS * COLS
NC, NS = 2, 16
NW = NC * NS
PER_W = TOTAL // NW
CHUNK = 16384
NCHUNK = PER_W // CHUNK
NPAIR = NCHUNK // 2
GROUPS = CHUNK // 128

_MESH = plsc.VectorSubcoreMesh(core_axis_name="c", subcore_axis_name="s")


def _body(x_hbm, s_hbm, o_hbm, xb0, xb1, sb0, sb1, ob0, ob1,
          in0, in1, out0, out1):
    wid = lax.axis_index("s") * NC + lax.axis_index("c")
    base_w = wid * PER_W
    vec8 = lax.iota(jnp.int32, 16) * 8
    xbs = (xb0, xb1)
    sbs = (sb0, sb1)
    obs = (ob0, ob1)
    ins = (in0, in1)
    outs = (out0, out1)

    def start_in(c, b):
        src = pl.ds(base_w + c * CHUNK, CHUNK)
        pltpu.async_copy(s_hbm.at[src], sbs[b], ins[b])
        pltpu.async_copy(x_hbm.at[src], xbs[b], ins[b])

    def wait_in(b):
        pltpu.make_async_copy(s_hbm.at[pl.ds(0, CHUNK)], sbs[b], ins[b]).wait()
        pltpu.make_async_copy(x_hbm.at[pl.ds(0, CHUNK)], xbs[b], ins[b]).wait()

    def start_out(c, b):
        dst = pl.ds(base_w + c * CHUNK, CHUNK)
        pltpu.async_copy(obs[b], o_hbm.at[dst], outs[b])

    def wait_out(b):
        pltpu.make_async_copy(obs[b], o_hbm.at[pl.ds(0, CHUNK)], outs[b]).wait()

    def compute(b):
        sbuf, xbuf, obuf = sbs[b], xbs[b], obs[b]

        @pl.loop(0, GROUPS)
        def _group(gi):
            g0 = gi * 128
            idx = [vec8 + (g0 + p) for p in range(8)]
            s = [plsc.load_gather(sbuf, [idx[p]]) for p in range(8)]
            for p in range(8):
                cnt = jnp.zeros((16,), jnp.int32)
                for q in range(8):
                    if q == p:
                        continue
                    below = (s[q] <= s[p]) if q < p else (s[q] < s[p])
                    cnt = cnt + jnp.where(below, 1, 0)
                keep = cnt >= 4
                xv = plsc.load_gather(xbuf, [idx[p]])
                plsc.store_scatter(obuf, [idx[p]], jnp.where(keep, xv, 0.0))

    start_in(0, 0)
    start_in(1, 1)

    @pl.loop(0, NPAIR)
    def _pair(ci2):
        for b in range(2):
            c = ci2 * 2 + b
            wait_in(b)

            @pl.when(ci2 >= 1)
            def _():
                wait_out(b)

            compute(b)
            start_out(c, b)

            @pl.when(ci2 <= NPAIR - 2)
            def _():
                start_in(c + 2, b)

    wait_out(0)
    wait_out(1)


@jax.jit
def _run(xf, sf):
    return pl.kernel(
        _body,
        out_type=jax.ShapeDtypeStruct((TOTAL,), jnp.float32),
        mesh=_MESH,
        scratch_types=[
            pltpu.VMEM((CHUNK,), jnp.float32),
            pltpu.VMEM((CHUNK,), jnp.float32),
            pltpu.VMEM((CHUNK,), jnp.float32),
            pltpu.VMEM((CHUNK,), jnp.float32),
            pltpu.VMEM((CHUNK,), jnp.float32),
            pltpu.VMEM((CHUNK,), jnp.float32),
            pltpu.SemaphoreType.DMA,
            pltpu.SemaphoreType.DMA,
            pltpu.SemaphoreType.DMA,
            pltpu.SemaphoreType.DMA,
        ],
        compiler_params=pltpu.CompilerParams(needs_layout_passes=False),
    )(xf, sf)


def kernel(x, score):
    out = _run(x.reshape(TOTAL), score.reshape(TOTAL))
    return out.reshape(ROWS, COLS)
